# baseline (speedup 1.0000x reference)
import jax
import jax.numpy as jnp
from jax.experimental import pallas as pl
from jax.experimental.pallas import tpu as pltpu

_NUM_BINS = 256
_MIN_VAL = 0.0
_MAX_VAL = 1.0
_LANES = 128
_ROWS_PER_STEP = 8


def _hist_kernel(x_ref, o_ref, acc_ref):
    inv_bw = (_NUM_BINS - 1) / (_MAX_VAL - _MIN_VAL)
    rows = x_ref.shape[1]
    bins = jax.lax.broadcasted_iota(
        jnp.int32, (_NUM_BINS, _ROWS_PER_STEP, _LANES), 0).astype(jnp.float32)
    acc_ref[...] = jnp.zeros_like(acc_ref)

    def body(i, carry):
        x = x_ref[0, pl.ds(i * _ROWS_PER_STEP, _ROWS_PER_STEP), :]
        t = (x - _MIN_VAL) * inv_bw
        w = jnp.maximum(1.0 - jnp.abs(jnp.broadcast_to(t[None, :, :], bins.shape) - bins), 0.0)
        acc_ref[...] += jnp.sum(w, axis=1)
        return carry

    jax.lax.fori_loop(0, rows // _ROWS_PER_STEP, body, 0, unroll=2)
    o_ref[...] = jnp.sum(acc_ref[...], axis=1).reshape(1, 1, _NUM_BINS)


def kernel(images_batch, bin_centers):
    del bin_centers
    b = images_batch.shape[0]
    n = images_batch.shape[1] * images_batch.shape[2] * images_batch.shape[3]
    rows = n // _LANES
    x = images_batch.reshape(b, rows, _LANES)
    out = pl.pallas_call(
        _hist_kernel,
        out_shape=jax.ShapeDtypeStruct((b, 1, _NUM_BINS), jnp.float32),
        grid=(b,),
        in_specs=[pl.BlockSpec((1, rows, _LANES), lambda i: (i, 0, 0))],
        out_specs=pl.BlockSpec((1, 1, _NUM_BINS), lambda i: (i, 0, 0)),
        scratch_shapes=[pltpu.VMEM((_NUM_BINS, _LANES), jnp.float32)],
        compiler_params=pltpu.CompilerParams(
            dimension_semantics=("parallel",),
        ),
    )(x)
    return out.reshape(b, _NUM_BINS)


# speedup vs baseline: 2.3958x; 2.3958x over previous
import jax
import jax.numpy as jnp
from jax.experimental import pallas as pl
from jax.experimental.pallas import tpu as pltpu

_NUM_BINS = 256
_MIN_VAL = 0.0
_MAX_VAL = 1.0
_LANES = 128
_BINS_PER_PASS = 128
_ROWS_PER_STEP = 8


def _hist_kernel(x_ref, o_ref):
    inv_bw = (_NUM_BINS - 1) / (_MAX_VAL - _MIN_VAL)
    rows = x_ref.shape[1]
    n_pixels = rows * _LANES
    n_slabs = rows // _ROWS_PER_STEP

    parts = []
    for bin_base in range(0, _NUM_BINS, _BINS_PER_PASS):
        bins = (jax.lax.broadcasted_iota(
            jnp.int32, (_BINS_PER_PASS, _LANES), 0).astype(jnp.float32)
            + float(bin_base))

        def body(i, acc, bins=bins):
            slab = x_ref[0, pl.ds(i * _ROWS_PER_STEP, _ROWS_PER_STEP), :]
            t = (slab - _MIN_VAL) * inv_bw
            for s in range(_ROWS_PER_STEP):
                bx = jnp.broadcast_to(t[s:s + 1, :], (_BINS_PER_PASS, _LANES))
                acc = acc + jnp.minimum(jnp.abs(bx - bins), 1.0)
            return acc

        acc0 = jnp.zeros((_BINS_PER_PASS, _LANES), jnp.float32)
        acc = jax.lax.fori_loop(0, n_slabs, body, acc0)
        parts.append(jnp.sum(acc, axis=1))

    total = jnp.concatenate(parts)
    o_ref[...] = (float(n_pixels) - total).reshape(1, 1, _NUM_BINS)


def kernel(images_batch, bin_centers):
    del bin_centers
    b = images_batch.shape[0]
    n = images_batch.shape[1] * images_batch.shape[2] * images_batch.shape[3]
    rows = n // _LANES
    x = images_batch.reshape(b, rows, _LANES)
    out = pl.pallas_call(
        _hist_kernel,
        out_shape=jax.ShapeDtypeStruct((b, 1, _NUM_BINS), jnp.float32),
        grid=(b,),
        in_specs=[pl.BlockSpec((1, rows, _LANES), lambda i: (i, 0, 0))],
        out_specs=pl.BlockSpec((1, 1, _NUM_BINS), lambda i: (i, 0, 0)),
        compiler_params=pltpu.CompilerParams(
            dimension_semantics=("parallel",),
        ),
    )(x)
    return out.reshape(b, _NUM_BINS)
